# Initial kernel scaffold; baseline (speedup 1.0000x reference)
import jax
import jax.numpy as jnp
from jax.experimental import pallas as pl


def kernel(features, prototypes):
    raise NotImplementedError("write your pallas kernel here")


# speedup vs baseline: 1.3134x; 1.3134x over previous
import jax
import jax.numpy as jnp
from jax import lax
from jax.experimental import pallas as pl
from jax.experimental.pallas import tpu as pltpu
from jax.experimental.pallas import tpu_sc as plsc

UPDATE_RATE = 0.1
EPS = 1e-8

_NC = 2
_NS = 16
_NW = _NC * _NS
_NG = 4
_CH = 128
_LN = 16


def _assign_body(f_ref, p_ref, out_ref):
    f = f_ref[...]
    p = p_ref[...]
    fn = f / jnp.maximum(jnp.sqrt(jnp.sum(f * f, axis=1, keepdims=True)), EPS)
    pn = p / jnp.maximum(jnp.sqrt(jnp.sum(p * p, axis=1, keepdims=True)), EPS)
    sims = lax.dot_general(
        fn, pn, (((1,), (1,)), ((), ())),
        preferred_element_type=jnp.float32,
    )
    m = jnp.max(sims, axis=1, keepdims=True)
    k = sims.shape[1]
    iota = lax.broadcasted_iota(jnp.int32, sims.shape, 1)
    idx = jnp.min(jnp.where(sims >= m, iota, k), axis=1)
    out_ref[...] = idx.reshape(1, 1, -1)


def _ema_body(p_ref, s_ref, a_ref, o_ref):
    k, d = o_ref.shape
    dg = d // _NG
    p = p_ref[...]
    kio = lax.broadcasted_iota(jnp.int32, (k, 1), 0)
    c = jnp.zeros((k, 1), jnp.float32)
    for b in range(a_ref.shape[0]):
        row = a_ref[b]
        c = c + jnp.sum((kio == row).astype(jnp.float32), axis=1, keepdims=True)
    groups = []
    for g in range(_NG):
        sg = s_ref[g]
        for m in range(1, _NW // _NG):
            sg = sg + s_ref[m * _NG + g]
        groups.append(jnp.concatenate([sg[:, :dg], sg[:, dg:]], axis=0))
    s = jnp.concatenate(groups, axis=1)
    mean = s / jnp.maximum(c, 1.0)
    o_ref[...] = jnp.where(c > 0, (1.0 - UPDATE_RATE) * p + UPDATE_RATE * mean, p)


def _sc_scatter_body(feat_hbm, asn_hbm, zeros_hbm, out_hbm, idx_v, rows_v, acc_v):
    n, d = feat_hbm.shape
    k2 = zeros_hbm.shape[0]
    dg = d // _NG
    cid = lax.axis_index("c")
    sid = lax.axis_index("s")
    wid = cid * _NS + sid
    g = lax.rem(wid, _NG)
    m = lax.div(wid, _NG)
    rows_per_member = n // (_NW // _NG)
    dh = lax.div(g, 2)
    inlane = lax.rem(g, 2) * dg

    pltpu.sync_copy(zeros_hbm, acc_v)

    for c in range(rows_per_member // _CH):
        base = m * rows_per_member + c * _CH
        pltpu.sync_copy(asn_hbm.at[pl.ds(base, _CH)], idx_v)
        pltpu.sync_copy(feat_hbm.at[pl.ds(base, _CH), pl.ds(dh * 128, 128)], rows_v)

        def row_body(i, carry):
            kvec = idx_v[pl.ds(i * _LN, _LN)]
            arow = lax.rem(kvec, k2)
            alane = lax.div(kvec, k2) * dg
            for l in range(_LN):
                rk = arow[l]
                lk = alane[l]
                r = i * _LN + l
                for j in range(dg // _LN):
                    vals = rows_v[r, pl.ds(inlane + j * _LN, _LN)]
                    plsc.addupdate(acc_v.at[rk, pl.ds(lk + j * _LN, _LN)], vals)
            return carry

        lax.fori_loop(0, _CH // _LN, row_body, 0)

    pltpu.sync_copy(acc_v, out_hbm.at[wid])


def _make_sc_scatter(n, d, k):
    mesh = plsc.VectorSubcoreMesh(core_axis_name="c", subcore_axis_name="s")
    return pl.kernel(
        _sc_scatter_body,
        out_type=jax.ShapeDtypeStruct((_NW, k // 2, d // 2), jnp.float32),
        mesh=mesh,
        scratch_types=[
            pltpu.VMEM((_CH,), jnp.int32),
            pltpu.VMEM((_CH, d // 2), jnp.float32),
            pltpu.VMEM((k // 2, d // 2), jnp.float32),
        ],
    )


@jax.jit
def kernel(features, prototypes):
    n, d = features.shape
    k = prototypes.shape[0]
    bn = 512
    grid = n // bn

    asn3 = pl.pallas_call(
        _assign_body,
        grid=(grid,),
        in_specs=[
            pl.BlockSpec((bn, d), lambda i: (i, 0)),
            pl.BlockSpec((k, d), lambda i: (0, 0)),
        ],
        out_specs=pl.BlockSpec((1, 1, bn), lambda i: (i, 0, 0)),
        out_shape=jax.ShapeDtypeStruct((grid, 1, bn), jnp.int32),
    )(features, prototypes)
    assignments = asn3.reshape(n)

    zeros = jnp.zeros((k // 2, d // 2), jnp.float32)
    sums = _make_sc_scatter(n, d, k)(features, assignments, zeros)

    updated = pl.pallas_call(
        _ema_body,
        in_specs=[
            pl.BlockSpec((k, d), lambda: (0, 0)),
            pl.BlockSpec((_NW, k // 2, d // 2), lambda: (0, 0, 0)),
            pl.BlockSpec((grid, 1, bn), lambda: (0, 0, 0)),
        ],
        out_specs=pl.BlockSpec((k, d), lambda: (0, 0)),
        out_shape=jax.ShapeDtypeStruct((k, d), jnp.float32),
    )(prototypes, sums, asn3)
    return updated


# speedup vs baseline: 1.6258x; 1.2378x over previous
import jax
import jax.numpy as jnp
from jax import lax
from jax.experimental import pallas as pl
from jax.experimental.pallas import tpu as pltpu
from jax.experimental.pallas import tpu_sc as plsc

UPDATE_RATE = 0.1
EPS = 1e-8

_NC = 2
_NS = 16
_NW = _NC * _NS
_NG = 4
_CH = 128
_ICH = 512
_LN = 16


def _assign_body(f_ref, p_ref, out_ref, pn_ref):
    @pl.when(pl.program_id(0) == 0)
    def _():
        p = p_ref[...]
        pn_ref[...] = p / jnp.maximum(
            jnp.sqrt(jnp.sum(p * p, axis=1, keepdims=True)), EPS)

    f = f_ref[...]
    fn = f / jnp.maximum(jnp.sqrt(jnp.sum(f * f, axis=1, keepdims=True)), EPS)
    sims = lax.dot_general(
        fn, pn_ref[...], (((1,), (1,)), ((), ())),
        preferred_element_type=jnp.float32,
    )
    m = jnp.max(sims, axis=1, keepdims=True)
    k = sims.shape[1]
    iota = lax.broadcasted_iota(jnp.int32, sims.shape, 1)
    idx = jnp.min(jnp.where(sims >= m, iota, k), axis=1)
    out_ref[...] = idx.reshape(1, 1, -1)


def _ema_body(p_ref, s_ref, a_ref, o_ref):
    k, d = o_ref.shape
    dg = d // _NG
    p = p_ref[...]
    kio = lax.broadcasted_iota(jnp.int32, (k, 1), 0)
    c = jnp.zeros((k, 1), jnp.float32)
    for b in range(a_ref.shape[0]):
        row = a_ref[b]
        c = c + jnp.sum((kio == row).astype(jnp.float32), axis=1, keepdims=True)
    groups = []
    for g in range(_NG):
        sg = s_ref[g]
        for m in range(1, _NW // _NG):
            sg = sg + s_ref[m * _NG + g]
        groups.append(jnp.concatenate([sg[:, :dg], sg[:, dg:]], axis=0))
    s = jnp.concatenate(groups, axis=1)
    mean = s / jnp.maximum(c, 1.0)
    o_ref[...] = jnp.where(c > 0, (1.0 - UPDATE_RATE) * p + UPDATE_RATE * mean, p)


def _sc_scatter_body(feat_hbm, asn_hbm, zeros_hbm, out_hbm,
                     idx_v, rows_a, rows_b, acc_v, sem_a, sem_b):
    n, d = feat_hbm.shape
    k2 = zeros_hbm.shape[0]
    dg = d // _NG
    kshift = (k2 - 1).bit_length()
    cid = lax.axis_index("c")
    sid = lax.axis_index("s")
    wid = cid * _NS + sid
    g = lax.rem(wid, _NG)
    m = lax.div(wid, _NG)
    rows_per_member = n // (_NW // _NG)
    dh = lax.div(g, 2)
    inlane = lax.rem(g, 2) * dg
    rbase = m * rows_per_member

    pltpu.sync_copy(zeros_hbm, acc_v)

    bufs = (rows_a, rows_b)
    sems = (sem_a, sem_b)

    def start(c):
        return pltpu.async_copy(
            feat_hbm.at[pl.ds(rbase + c * _CH, _CH), pl.ds(dh * 128, 128)],
            bufs[c % 2], sems[c % 2])

    n_chunks = rows_per_member // _CH
    cp = start(0)
    for c in range(n_chunks):
        if c % (_ICH // _CH) == 0:
            pltpu.sync_copy(asn_hbm.at[pl.ds(rbase + c * _CH, _ICH)], idx_v)
        ioff = (c % (_ICH // _CH)) * _CH
        cp.wait()
        if c + 1 < n_chunks:
            cp = start(c + 1)
        rows_v = bufs[c % 2]

        def blk16(q, carry):
            kvec = idx_v[pl.ds(ioff + q * _LN, _LN)]
            pkv = lax.shift_left(lax.bitwise_and(kvec, k2 - 1), 7) + \
                lax.shift_left(lax.shift_right_logical(kvec, kshift), 6)
            for u in range(_LN):
                pk = pkv[u]
                arow = lax.shift_right_logical(pk, 7)
                alane = lax.bitwise_and(pk, 127)
                i = q * _LN + u
                for j in range(dg // _LN):
                    vals = rows_v[i, pl.ds(inlane + j * _LN, _LN)]
                    plsc.addupdate(acc_v.at[arow, pl.ds(alane + j * _LN, _LN)], vals)
            return carry

        lax.fori_loop(0, _CH // _LN, blk16, 0)

    pltpu.sync_copy(acc_v, out_hbm.at[wid])


def _make_sc_scatter(n, d, k):
    mesh = plsc.VectorSubcoreMesh(core_axis_name="c", subcore_axis_name="s")
    return pl.kernel(
        _sc_scatter_body,
        out_type=jax.ShapeDtypeStruct((_NW, k // 2, d // 2), jnp.float32),
        mesh=mesh,
        scratch_types=[
            pltpu.VMEM((_ICH,), jnp.int32),
            pltpu.VMEM((_CH, d // 2), jnp.float32),
            pltpu.VMEM((_CH, d // 2), jnp.float32),
            pltpu.VMEM((k // 2, d // 2), jnp.float32),
            pltpu.SemaphoreType.DMA,
            pltpu.SemaphoreType.DMA,
        ],
    )


@jax.jit
def kernel(features, prototypes):
    n, d = features.shape
    k = prototypes.shape[0]
    bn = 512
    grid = n // bn

    asn3 = pl.pallas_call(
        _assign_body,
        grid=(grid,),
        in_specs=[
            pl.BlockSpec((bn, d), lambda i: (i, 0)),
            pl.BlockSpec((k, d), lambda i: (0, 0)),
        ],
        out_specs=pl.BlockSpec((1, 1, bn), lambda i: (i, 0, 0)),
        out_shape=jax.ShapeDtypeStruct((grid, 1, bn), jnp.int32),
        scratch_shapes=[pltpu.VMEM((k, d), jnp.float32)],
    )(features, prototypes)
    assignments = asn3.reshape(n)

    zeros = jnp.zeros((k // 2, d // 2), jnp.float32)
    sums = _make_sc_scatter(n, d, k)(features, assignments, zeros)

    updated = pl.pallas_call(
        _ema_body,
        in_specs=[
            pl.BlockSpec((k, d), lambda: (0, 0)),
            pl.BlockSpec((_NW, k // 2, d // 2), lambda: (0, 0, 0)),
            pl.BlockSpec((grid, 1, bn), lambda: (0, 0, 0)),
        ],
        out_specs=pl.BlockSpec((k, d), lambda: (0, 0)),
        out_shape=jax.ShapeDtypeStruct((k, d), jnp.float32),
    )(prototypes, sums, asn3)
    return updated


# speedup vs baseline: 1.6273x; 1.0009x over previous
import jax
import jax.numpy as jnp
from jax import lax
from jax.experimental import pallas as pl
from jax.experimental.pallas import tpu as pltpu
from jax.experimental.pallas import tpu_sc as plsc

UPDATE_RATE = 0.1
EPS = 1e-8

_NC = 2
_NS = 16
_NW = _NC * _NS
_NG = 4
_CH = 128
_ICH = 512
_LN = 16


def _assign_body(f_ref, p_ref, out_ref, pn_ref):
    @pl.when(pl.program_id(0) == 0)
    def _():
        p = p_ref[...]
        pn_ref[...] = p / jnp.maximum(
            jnp.sqrt(jnp.sum(p * p, axis=1, keepdims=True)), EPS)

    f = f_ref[...]
    fn = f / jnp.maximum(jnp.sqrt(jnp.sum(f * f, axis=1, keepdims=True)), EPS)
    sims = lax.dot_general(
        fn, pn_ref[...], (((1,), (1,)), ((), ())),
        preferred_element_type=jnp.float32,
    )
    m = jnp.max(sims, axis=1, keepdims=True)
    k = sims.shape[1]
    iota = lax.broadcasted_iota(jnp.int32, sims.shape, 1)
    idx = jnp.min(jnp.where(sims >= m, iota, k), axis=1)
    out_ref[...] = idx.reshape(1, 1, -1)


def _ema_body(p_ref, s_ref, a_ref, o_ref):
    k, d = o_ref.shape
    dg = d // _NG
    p = p_ref[...]
    kio = lax.broadcasted_iota(jnp.int32, (k, 1), 0)
    c = jnp.zeros((k, 1), jnp.float32)
    for b in range(a_ref.shape[0]):
        row = a_ref[b]
        c = c + jnp.sum((kio == row).astype(jnp.float32), axis=1, keepdims=True)
    groups = []
    for g in range(_NG):
        sg = s_ref[g]
        for m in range(1, _NW // _NG):
            sg = sg + s_ref[m * _NG + g]
        groups.append(jnp.concatenate([sg[:, :dg], sg[:, dg:]], axis=0))
    s = jnp.concatenate(groups, axis=1)
    mean = s / jnp.maximum(c, 1.0)
    o_ref[...] = jnp.where(c > 0, (1.0 - UPDATE_RATE) * p + UPDATE_RATE * mean, p)


def _sc_scatter_body(half, nh, feat_hbm, asn_hbm, zeros_hbm, out_hbm,
                     idx_v, rows_a, rows_b, acc_v, sem_a, sem_b):
    n, d = feat_hbm.shape
    k2 = zeros_hbm.shape[0]
    dg = d // _NG
    kshift = (k2 - 1).bit_length()
    cid = lax.axis_index("c")
    sid = lax.axis_index("s")
    wid = cid * _NS + sid
    g = lax.rem(wid, _NG)
    m = lax.div(wid, _NG)
    rows_per_member = nh // (_NW // _NG)
    dh = lax.div(g, 2)
    inlane = lax.rem(g, 2) * dg
    rbase = half * nh + m * rows_per_member

    pltpu.sync_copy(zeros_hbm, acc_v)

    bufs = (rows_a, rows_b)
    sems = (sem_a, sem_b)

    def start(c):
        return pltpu.async_copy(
            feat_hbm.at[pl.ds(rbase + c * _CH, _CH), pl.ds(dh * 128, 128)],
            bufs[c % 2], sems[c % 2])

    n_chunks = rows_per_member // _CH
    cp = start(0)
    for c in range(n_chunks):
        if c % (_ICH // _CH) == 0:
            pltpu.sync_copy(
                asn_hbm.at[pl.ds(rbase - half * nh + c * _CH, _ICH)], idx_v)
        ioff = (c % (_ICH // _CH)) * _CH
        cp.wait()
        if c + 1 < n_chunks:
            cp = start(c + 1)
        rows_v = bufs[c % 2]

        def blk16(q, carry):
            kvec = idx_v[pl.ds(ioff + q * _LN, _LN)]
            pkv = lax.shift_left(lax.bitwise_and(kvec, k2 - 1), 7) + \
                lax.shift_left(lax.shift_right_logical(kvec, kshift), 6)
            for u in range(_LN):
                pk = pkv[u]
                arow = lax.shift_right_logical(pk, 7)
                alane = lax.bitwise_and(pk, 127)
                i = q * _LN + u
                for j in range(dg // _LN):
                    vals = rows_v[i, pl.ds(inlane + j * _LN, _LN)]
                    plsc.addupdate(acc_v.at[arow, pl.ds(alane + j * _LN, _LN)], vals)
            return carry

        lax.fori_loop(0, _CH // _LN, blk16, 0)

    pltpu.sync_copy(acc_v, out_hbm.at[wid])


def _make_sc_scatter(n, d, k, half, nh):
    import functools as _ft
    mesh = plsc.VectorSubcoreMesh(core_axis_name="c", subcore_axis_name="s")
    return pl.kernel(
        _ft.partial(_sc_scatter_body, half, nh),
        out_type=jax.ShapeDtypeStruct((_NW, k // 2, d // 2), jnp.float32),
        mesh=mesh,
        scratch_types=[
            pltpu.VMEM((_ICH,), jnp.int32),
            pltpu.VMEM((_CH, d // 2), jnp.float32),
            pltpu.VMEM((_CH, d // 2), jnp.float32),
            pltpu.VMEM((k // 2, d // 2), jnp.float32),
            pltpu.SemaphoreType.DMA,
            pltpu.SemaphoreType.DMA,
        ],
    )


@jax.jit
def kernel(features, prototypes):
    n, d = features.shape
    k = prototypes.shape[0]
    bn = 512
    grid = n // bn

    asn3 = pl.pallas_call(
        _assign_body,
        grid=(grid,),
        in_specs=[
            pl.BlockSpec((bn, d), lambda i: (i, 0)),
            pl.BlockSpec((k, d), lambda i: (0, 0)),
        ],
        out_specs=pl.BlockSpec((1, 1, bn), lambda i: (i, 0, 0)),
        out_shape=jax.ShapeDtypeStruct((grid, 1, bn), jnp.int32),
        scratch_shapes=[pltpu.VMEM((k, d), jnp.float32)],
    )(features, prototypes)

    zeros = jnp.zeros((k // 2, d // 2), jnp.float32)
    sums = _make_sc_scatter(n, d, k, 0, n)(features, asn3.reshape(n), zeros)

    updated = pl.pallas_call(
        _ema_body,
        in_specs=[
            pl.BlockSpec((k, d), lambda: (0, 0)),
            pl.BlockSpec((_NW, k // 2, d // 2), lambda: (0, 0, 0)),
            pl.BlockSpec((grid, 1, bn), lambda: (0, 0, 0)),
        ],
        out_specs=pl.BlockSpec((k, d), lambda: (0, 0)),
        out_shape=jax.ShapeDtypeStruct((k, d), jnp.float32),
    )(prototypes, sums, asn3)
    return updated
